# baseline (speedup 1.0000x reference)
import jax
import jax.numpy as jnp
from jax import lax
from jax.experimental import pallas as pl
from jax.experimental.pallas import tpu as pltpu
from jax.experimental.pallas import tpu_sc as plsc

N = 10000
E = 320000
D = 128

NC = 2
NS = 16
L = 16
NW = NC * NS

CHUNK = 64
CHUNKS_PER_W = 160
SLICE = 16
N_SLICES = CHUNKS_PER_W // SLICE
NBUF = 4
CHUNK_C = 128
CPW_C = 80
SLICE_C = 16
E_PAD = NW * CHUNKS_PER_W * CHUNK
RPS = 632
N_PAD = NS * RPS


def _fill_tile(ref, rows, cols, value):
    v = jnp.full((L,), value, jnp.float32)

    @pl.loop(0, rows)
    def _(r):
        @pl.loop(0, cols, step=L)
        def _(c):
            ref[r, pl.ds(c, L)] = v


def _make_sc_agg(d):
    mesh = plsc.VectorSubcoreMesh(core_axis_name="c", subcore_axis_name="s")
    out_type = jax.ShapeDtypeStruct((NC, N_PAD, d), jnp.float32)
    scratch = [
        pltpu.VMEM((SLICE, CHUNK), jnp.int32),
        pltpu.VMEM((SLICE, CHUNK), jnp.int32),
    ] + [
        pltpu.VMEM((CHUNK, d), jnp.float32)
        for _ in range(NBUF)
    ] + [
        pltpu.VMEM_SHARED((N_PAD, d), jnp.float32),
    ] + [pltpu.SemaphoreType.DMA for _ in range(NBUF)]

    def body(feats, srcs, dsts, agg_out, src_v, dst_v, *rest):
        bufs = rest[:NBUF]
        acc = rest[NBUF]
        sems = rest[NBUF + 1:]
        cid = lax.axis_index("c")
        sid = lax.axis_index("s")
        wid = cid * NS + sid

        nfull, rem = RPS // CHUNK, RPS % CHUNK
        _fill_tile(bufs[0], CHUNK, d, 0.0)
        zbase = sid * RPS

        @pl.loop(0, nfull)
        def _(i):
            pltpu.sync_copy(bufs[0], acc.at[pl.ds(zbase + i * CHUNK, CHUNK)])

        if rem:
            pltpu.sync_copy(bufs[0].at[pl.ds(0, rem)],
                            acc.at[pl.ds(zbase + nfull * CHUNK, rem)])
        plsc.subcore_barrier()

        def gather(j, b):
            return pltpu.make_async_copy(feats.at[src_v.at[j]], bufs[b],
                                         sems[b])

        for h in range(N_SLICES):
            pltpu.sync_copy(srcs.at[wid].at[pl.ds(h * SLICE, SLICE)], src_v)
            pltpu.sync_copy(dsts.at[wid].at[pl.ds(h * SLICE, SLICE)], dst_v)

            for b in range(NBUF):
                gather(b, b).start()

            @pl.loop(0, SLICE, step=NBUF)
            def _(jj):
                for b in range(NBUF):
                    gather(jj + b, b).wait()
                    pltpu.sync_copy(bufs[b], acc.at[dst_v.at[jj + b]],
                                    add=True)

                    @pl.when(jj + b + NBUF < SLICE)
                    def _():
                        gather(jj + b + NBUF, b).start()

        plsc.subcore_barrier()

        obase = sid * RPS
        pltpu.sync_copy(acc.at[pl.ds(obase, RPS)],
                        agg_out.at[cid].at[pl.ds(obase, RPS)])

    return pl.kernel(body, out_type=out_type, mesh=mesh,
                     scratch_types=scratch)


def _make_sc_counts():
    mesh = plsc.VectorSubcoreMesh(core_axis_name="c", subcore_axis_name="s")
    out_type = jax.ShapeDtypeStruct((NC, N_PAD, D), jnp.float32)
    scratch = [
        pltpu.VMEM((SLICE_C, CHUNK_C), jnp.int32),
        pltpu.VMEM((CHUNK_C, D), jnp.float32),
        pltpu.VMEM_SHARED((N_PAD, D), jnp.float32),
    ]

    def body(dsts, cnt_out, dst_v, ones_t, acc):
        cid = lax.axis_index("c")
        sid = lax.axis_index("s")
        wid = cid * NS + sid

        nfull, rem = RPS // CHUNK_C, RPS % CHUNK_C
        _fill_tile(ones_t, CHUNK_C, D, 0.0)
        zbase = sid * RPS

        @pl.loop(0, nfull)
        def _(i):
            pltpu.sync_copy(ones_t, acc.at[pl.ds(zbase + i * CHUNK_C, CHUNK_C)])

        if rem:
            pltpu.sync_copy(ones_t.at[pl.ds(0, rem)],
                            acc.at[pl.ds(zbase + nfull * CHUNK_C, rem)])
        _fill_tile(ones_t, CHUNK_C, D, 1.0)
        plsc.subcore_barrier()

        for h in range(CPW_C // SLICE_C):
            pltpu.sync_copy(dsts.at[wid].at[pl.ds(h * SLICE_C, SLICE_C)],
                            dst_v)

            @pl.loop(0, SLICE_C)
            def _(jj):
                pltpu.sync_copy(ones_t, acc.at[dst_v.at[jj]], add=True)

        plsc.subcore_barrier()
        obase = sid * RPS
        pltpu.sync_copy(acc.at[pl.ds(obase, RPS)],
                        cnt_out.at[cid].at[pl.ds(obase, RPS)])

    return pl.kernel(body, out_type=out_type, mesh=mesh,
                     scratch_types=scratch)


def _affine_body(x_ref, w_ref, b_ref, o_ref):
    o_ref[...] = lax.dot_general(
        x_ref[...], w_ref[...], (((1,), (1,)), ((), ())),
        preferred_element_type=jnp.float32) + b_ref[...]


_tc_affine = pl.pallas_call(
    _affine_body,
    out_shape=jax.ShapeDtypeStruct((N, D), jnp.float32),
)


def _inv_from_cnt(cnt_ref):
    cnt = cnt_ref[0][:N, 0:1] + cnt_ref[1][:N, 0:1]
    return 1.0 / jnp.maximum(cnt, 1.0)


def _mid_body(agg_ref, cnt_ref, z1_ref, w1l_ref, w2r_ref, b2_ref,
              h_ref, z2_ref):
    inv = _inv_from_cnt(cnt_ref)
    aggm = (agg_ref[0][:N] + agg_ref[1][:N]) * inv
    h = jnp.maximum(
        lax.dot_general(aggm, w1l_ref[...], (((1,), (1,)), ((), ())),
                        preferred_element_type=jnp.float32) + z1_ref[...],
        0.0)
    h_ref[...] = h
    z2_ref[...] = lax.dot_general(
        h, w2r_ref[...], (((1,), (1,)), ((), ())),
        preferred_element_type=jnp.float32) + b2_ref[...]


_tc_mid = pl.pallas_call(
    _mid_body,
    out_shape=[jax.ShapeDtypeStruct((N, D), jnp.float32),
               jax.ShapeDtypeStruct((N, D), jnp.float32)],
)


def _final_body(agg_ref, cnt_ref, z2_ref, w2l_ref, wlin_ref, blin_ref,
                out_ref):
    inv = _inv_from_cnt(cnt_ref)
    aggm = (agg_ref[0][:N] + agg_ref[1][:N]) * inv
    h2 = jnp.maximum(
        lax.dot_general(aggm, w2l_ref[...], (((1,), (1,)), ((), ())),
                        preferred_element_type=jnp.float32) + z2_ref[...],
        0.0)
    out_ref[...] = lax.dot_general(
        h2, wlin_ref[...], (((1,), (1,)), ((), ())),
        preferred_element_type=jnp.float32) + blin_ref[...]


_tc_final = pl.pallas_call(
    _final_body,
    out_shape=jax.ShapeDtypeStruct((N, 2), jnp.float32),
)


@jax.jit
def kernel(x, edge_index, W1_l, b1_l, W1_r, W2_l, b2_l, W2_r, W_lin, b_lin):
    sc_agg = _make_sc_agg(D)
    sc_counts = _make_sc_counts()

    src = edge_index[0].astype(jnp.int32)
    dst = edge_index[1].astype(jnp.int32)
    pad = E_PAD - E
    srcs = jnp.concatenate([src, jnp.zeros((pad,), jnp.int32)])
    dsts = jnp.concatenate([dst, jnp.full((pad,), N, jnp.int32)])
    srcs = srcs.reshape(NW, CHUNKS_PER_W, CHUNK)
    dsts_c = dsts.reshape(NW, CPW_C, CHUNK_C)
    dsts = dsts.reshape(NW, CHUNKS_PER_W, CHUNK)

    cnt = sc_counts(dsts_c)
    agg1 = sc_agg(x, srcs, dsts)
    z1 = _tc_affine(x, W1_r, b1_l.reshape(1, D))

    h, z2 = _tc_mid(agg1, cnt, z1, W1_l, W2_r, b2_l.reshape(1, D))

    agg2 = sc_agg(h, srcs, dsts)

    return _tc_final(agg2, cnt, z2, W2_l, W_lin, b_lin.reshape(1, 2))


# speedup vs baseline: 1.0642x; 1.0642x over previous
import jax
import jax.numpy as jnp
from jax import lax
from jax.experimental import pallas as pl
from jax.experimental.pallas import tpu as pltpu
from jax.experimental.pallas import tpu_sc as plsc

N = 10000
E = 320000
D = 128

NC = 2
NS = 16
L = 16
NW = NC * NS

CHUNK = 128
CHUNKS_PER_W = 80
SLICE = 16
N_SLICES = CHUNKS_PER_W // SLICE
NBUF = 2
CHUNK_C = 128
CPW_C = 80
SLICE_C = 16
E_PAD = NW * CHUNKS_PER_W * CHUNK
RPS = 632
N_PAD = NS * RPS


def _fill_tile(ref, rows, cols, value):
    v = jnp.full((L,), value, jnp.float32)

    @pl.loop(0, rows)
    def _(r):
        @pl.loop(0, cols, step=L)
        def _(c):
            ref[r, pl.ds(c, L)] = v


def _make_sc_agg(d):
    mesh = plsc.VectorSubcoreMesh(core_axis_name="c", subcore_axis_name="s")
    out_type = jax.ShapeDtypeStruct((NC, N_PAD, d), jnp.float32)
    scratch = [
        pltpu.VMEM((SLICE, CHUNK), jnp.int32),
        pltpu.VMEM((SLICE, CHUNK), jnp.int32),
    ] + [
        pltpu.VMEM((CHUNK, d), jnp.float32)
        for _ in range(NBUF)
    ] + [
        pltpu.VMEM_SHARED((N_PAD, d), jnp.float32),
    ] + [pltpu.SemaphoreType.DMA for _ in range(NBUF)]

    def body(feats, srcs, dsts, agg_out, src_v, dst_v, *rest):
        bufs = rest[:NBUF]
        acc = rest[NBUF]
        sems = rest[NBUF + 1:]
        cid = lax.axis_index("c")
        sid = lax.axis_index("s")
        wid = cid * NS + sid

        nfull, rem = RPS // CHUNK, RPS % CHUNK
        _fill_tile(bufs[0], CHUNK, d, 0.0)
        zbase = sid * RPS

        @pl.loop(0, nfull)
        def _(i):
            pltpu.sync_copy(bufs[0], acc.at[pl.ds(zbase + i * CHUNK, CHUNK)])

        if rem:
            pltpu.sync_copy(bufs[0].at[pl.ds(0, rem)],
                            acc.at[pl.ds(zbase + nfull * CHUNK, rem)])
        plsc.subcore_barrier()

        def gather(j, b):
            return pltpu.make_async_copy(feats.at[src_v.at[j]], bufs[b],
                                         sems[b])

        for h in range(N_SLICES):
            pltpu.sync_copy(srcs.at[wid].at[pl.ds(h * SLICE, SLICE)], src_v)
            pltpu.sync_copy(dsts.at[wid].at[pl.ds(h * SLICE, SLICE)], dst_v)

            for b in range(NBUF):
                gather(b, b).start()

            @pl.loop(0, SLICE, step=NBUF)
            def _(jj):
                for b in range(NBUF):
                    gather(jj + b, b).wait()
                    pltpu.sync_copy(bufs[b], acc.at[dst_v.at[jj + b]],
                                    add=True)

                    @pl.when(jj + b + NBUF < SLICE)
                    def _():
                        gather(jj + b + NBUF, b).start()

        plsc.subcore_barrier()

        obase = sid * RPS
        pltpu.sync_copy(acc.at[pl.ds(obase, RPS)],
                        agg_out.at[cid].at[pl.ds(obase, RPS)])

    return pl.kernel(body, out_type=out_type, mesh=mesh,
                     scratch_types=scratch)


def _make_sc_counts():
    mesh = plsc.VectorSubcoreMesh(core_axis_name="c", subcore_axis_name="s")
    out_type = jax.ShapeDtypeStruct((NC, N_PAD, D), jnp.float32)
    scratch = [
        pltpu.VMEM((SLICE_C, CHUNK_C), jnp.int32),
        pltpu.VMEM((CHUNK_C, D), jnp.float32),
        pltpu.VMEM_SHARED((N_PAD, D), jnp.float32),
    ]

    def body(dsts, cnt_out, dst_v, ones_t, acc):
        cid = lax.axis_index("c")
        sid = lax.axis_index("s")
        wid = cid * NS + sid

        nfull, rem = RPS // CHUNK_C, RPS % CHUNK_C
        _fill_tile(ones_t, CHUNK_C, D, 0.0)
        zbase = sid * RPS

        @pl.loop(0, nfull)
        def _(i):
            pltpu.sync_copy(ones_t, acc.at[pl.ds(zbase + i * CHUNK_C, CHUNK_C)])

        if rem:
            pltpu.sync_copy(ones_t.at[pl.ds(0, rem)],
                            acc.at[pl.ds(zbase + nfull * CHUNK_C, rem)])
        _fill_tile(ones_t, CHUNK_C, D, 1.0)
        plsc.subcore_barrier()

        for h in range(CPW_C // SLICE_C):
            pltpu.sync_copy(dsts.at[wid].at[pl.ds(h * SLICE_C, SLICE_C)],
                            dst_v)

            @pl.loop(0, SLICE_C)
            def _(jj):
                pltpu.sync_copy(ones_t, acc.at[dst_v.at[jj]], add=True)

        plsc.subcore_barrier()
        obase = sid * RPS
        pltpu.sync_copy(acc.at[pl.ds(obase, RPS)],
                        cnt_out.at[cid].at[pl.ds(obase, RPS)])

    return pl.kernel(body, out_type=out_type, mesh=mesh,
                     scratch_types=scratch)


def _affine_body(x_ref, w_ref, b_ref, o_ref):
    o_ref[...] = lax.dot_general(
        x_ref[...], w_ref[...], (((1,), (1,)), ((), ())),
        preferred_element_type=jnp.float32) + b_ref[...]


_tc_affine = pl.pallas_call(
    _affine_body,
    out_shape=jax.ShapeDtypeStruct((N, D), jnp.float32),
)


def _inv_from_cnt(cnt_ref):
    cnt = cnt_ref[0][:N, 0:1] + cnt_ref[1][:N, 0:1]
    return 1.0 / jnp.maximum(cnt, 1.0)


def _mid_body(agg_ref, cnt_ref, z1_ref, w1l_ref, w2r_ref, b2_ref,
              h_ref, z2_ref):
    inv = _inv_from_cnt(cnt_ref)
    aggm = (agg_ref[0][:N] + agg_ref[1][:N]) * inv
    h = jnp.maximum(
        lax.dot_general(aggm, w1l_ref[...], (((1,), (1,)), ((), ())),
                        preferred_element_type=jnp.float32) + z1_ref[...],
        0.0)
    h_ref[...] = h
    z2_ref[...] = lax.dot_general(
        h, w2r_ref[...], (((1,), (1,)), ((), ())),
        preferred_element_type=jnp.float32) + b2_ref[...]


_tc_mid = pl.pallas_call(
    _mid_body,
    out_shape=[jax.ShapeDtypeStruct((N, D), jnp.float32),
               jax.ShapeDtypeStruct((N, D), jnp.float32)],
)


def _final_body(agg_ref, cnt_ref, z2_ref, w2l_ref, wlin_ref, blin_ref,
                out_ref):
    inv = _inv_from_cnt(cnt_ref)
    aggm = (agg_ref[0][:N] + agg_ref[1][:N]) * inv
    h2 = jnp.maximum(
        lax.dot_general(aggm, w2l_ref[...], (((1,), (1,)), ((), ())),
                        preferred_element_type=jnp.float32) + z2_ref[...],
        0.0)
    out_ref[...] = lax.dot_general(
        h2, wlin_ref[...], (((1,), (1,)), ((), ())),
        preferred_element_type=jnp.float32) + blin_ref[...]


_tc_final = pl.pallas_call(
    _final_body,
    out_shape=jax.ShapeDtypeStruct((N, 2), jnp.float32),
)


@jax.jit
def kernel(x, edge_index, W1_l, b1_l, W1_r, W2_l, b2_l, W2_r, W_lin, b_lin):
    sc_agg = _make_sc_agg(D)
    sc_counts = _make_sc_counts()

    src = edge_index[0].astype(jnp.int32)
    dst = edge_index[1].astype(jnp.int32)
    pad = E_PAD - E
    srcs = jnp.concatenate([src, jnp.zeros((pad,), jnp.int32)])
    dsts = jnp.concatenate([dst, jnp.full((pad,), N, jnp.int32)])
    srcs = srcs.reshape(NW, CHUNKS_PER_W, CHUNK)
    dsts_c = dsts.reshape(NW, CPW_C, CHUNK_C)
    dsts = dsts.reshape(NW, CHUNKS_PER_W, CHUNK)

    cnt = sc_counts(dsts_c)
    agg1 = sc_agg(x, srcs, dsts)
    z1 = _tc_affine(x, W1_r, b1_l.reshape(1, D))

    h, z2 = _tc_mid(agg1, cnt, z1, W1_l, W2_r, b2_l.reshape(1, D))

    agg2 = sc_agg(h, srcs, dsts)

    return _tc_final(agg2, cnt, z2, W2_l, W_lin, b_lin.reshape(1, 2))


# speedup vs baseline: 1.1766x; 1.1056x over previous
import jax
import jax.numpy as jnp
from jax import lax
from jax.experimental import pallas as pl
from jax.experimental.pallas import tpu as pltpu
from jax.experimental.pallas import tpu_sc as plsc

N = 10000
E = 320000
D = 128

NC = 2
NS = 16
L = 16
NW = NC * NS

CHUNK = 128
CHUNKS_PER_W = 80
SLICE = 16
N_SLICES = CHUNKS_PER_W // SLICE
NBUF = 2
CHUNK_C = 128
CPW_C = 80
SLICE_C = 16
E_PAD = NW * CHUNKS_PER_W * CHUNK
RPS = 632
N_PAD = NS * RPS


def _fill_tile(ref, rows, cols, value):
    v = jnp.full((L,), value, jnp.float32)

    @pl.loop(0, rows)
    def _(r):
        @pl.loop(0, cols, step=L)
        def _(c):
            ref[r, pl.ds(c, L)] = v


def _make_sc_agg(d):
    mesh = plsc.VectorSubcoreMesh(core_axis_name="c", subcore_axis_name="s")
    out_type = jax.ShapeDtypeStruct((NC, N_PAD, d), jnp.float32)
    scratch = [
        pltpu.VMEM((SLICE, CHUNK), jnp.int32),
        pltpu.VMEM((SLICE, CHUNK), jnp.int32),
    ] + [
        pltpu.VMEM((CHUNK, d), jnp.float32)
        for _ in range(NBUF)
    ] + [
        pltpu.VMEM_SHARED((N_PAD, d), jnp.float32),
    ] + [pltpu.SemaphoreType.DMA for _ in range(NBUF)]

    def body(feats, srcs, dsts, agg_out, src_v, dst_v, *rest):
        bufs = rest[:NBUF]
        acc = rest[NBUF]
        sems = rest[NBUF + 1:]
        cid = lax.axis_index("c")
        sid = lax.axis_index("s")
        wid = cid * NS + sid

        nfull, rem = RPS // CHUNK, RPS % CHUNK
        _fill_tile(bufs[0], CHUNK, d, 0.0)
        zbase = sid * RPS

        @pl.loop(0, nfull)
        def _(i):
            pltpu.sync_copy(bufs[0], acc.at[pl.ds(zbase + i * CHUNK, CHUNK)])

        if rem:
            pltpu.sync_copy(bufs[0].at[pl.ds(0, rem)],
                            acc.at[pl.ds(zbase + nfull * CHUNK, rem)])
        plsc.subcore_barrier()

        def gather(j, b):
            return pltpu.make_async_copy(feats.at[src_v.at[j]], bufs[b],
                                         sems[b])

        for h in range(N_SLICES):
            pltpu.sync_copy(srcs.at[wid].at[pl.ds(h * SLICE, SLICE)], src_v)
            pltpu.sync_copy(dsts.at[wid].at[pl.ds(h * SLICE, SLICE)], dst_v)

            for b in range(NBUF):
                gather(b, b).start()

            @pl.loop(0, SLICE, step=NBUF)
            def _(jj):
                for b in range(NBUF):
                    gather(jj + b, b).wait()
                    pltpu.sync_copy(bufs[b], acc.at[dst_v.at[jj + b]],
                                    add=True)

                    @pl.when(jj + b + NBUF < SLICE)
                    def _():
                        gather(jj + b + NBUF, b).start()

        plsc.subcore_barrier()

        obase = sid * RPS
        pltpu.sync_copy(acc.at[pl.ds(obase, RPS)],
                        agg_out.at[cid].at[pl.ds(obase, RPS)])

    return pl.kernel(body, out_type=out_type, mesh=mesh,
                     scratch_types=scratch)


def _make_sc_counts():
    mesh = plsc.VectorSubcoreMesh(core_axis_name="c", subcore_axis_name="s")
    out_type = jax.ShapeDtypeStruct((NC, N_PAD, D), jnp.float32)
    scratch = [
        pltpu.VMEM((SLICE_C, CHUNK_C), jnp.int32),
        pltpu.VMEM((CHUNK_C, D), jnp.float32),
        pltpu.VMEM_SHARED((N_PAD, D), jnp.float32),
    ]

    def body(dsts, cnt_out, dst_v, ones_t, acc):
        cid = lax.axis_index("c")
        sid = lax.axis_index("s")
        wid = cid * NS + sid

        nfull, rem = RPS // CHUNK_C, RPS % CHUNK_C
        _fill_tile(ones_t, CHUNK_C, D, 0.0)
        zbase = sid * RPS

        @pl.loop(0, nfull)
        def _(i):
            pltpu.sync_copy(ones_t, acc.at[pl.ds(zbase + i * CHUNK_C, CHUNK_C)])

        if rem:
            pltpu.sync_copy(ones_t.at[pl.ds(0, rem)],
                            acc.at[pl.ds(zbase + nfull * CHUNK_C, rem)])
        _fill_tile(ones_t, CHUNK_C, D, 1.0)
        plsc.subcore_barrier()

        for h in range(CPW_C // SLICE_C):
            pltpu.sync_copy(dsts.at[wid].at[pl.ds(h * SLICE_C, SLICE_C)],
                            dst_v)

            @pl.loop(0, SLICE_C)
            def _(jj):
                pltpu.sync_copy(ones_t, acc.at[dst_v.at[jj]], add=True)

        plsc.subcore_barrier()
        obase = sid * RPS
        pltpu.sync_copy(acc.at[pl.ds(obase, RPS)],
                        cnt_out.at[cid].at[pl.ds(obase, RPS)])

    return pl.kernel(body, out_type=out_type, mesh=mesh,
                     scratch_types=scratch)


def _copy_body(x_ref, o_ref):
    o_ref[...] = x_ref[...]


_tc_copy = pl.pallas_call(
    _copy_body,
    out_shape=jax.ShapeDtypeStruct((N, D), jnp.float32),
)


def _affine_body(x_ref, w_ref, b_ref, o_ref):
    o_ref[...] = lax.dot_general(
        x_ref[...], w_ref[...], (((1,), (1,)), ((), ())),
        preferred_element_type=jnp.float32) + b_ref[...]


_tc_affine = pl.pallas_call(
    _affine_body,
    out_shape=jax.ShapeDtypeStruct((N, D), jnp.float32),
)


def _inv_from_cnt(cnt_ref):
    cnt = cnt_ref[0][:N, 0:1] + cnt_ref[1][:N, 0:1]
    return 1.0 / jnp.maximum(cnt, 1.0)


def _mid_body(agg_ref, cnt_ref, z1_ref, w1l_ref, w2r_ref, b2_ref,
              h_ref, z2_ref):
    inv = _inv_from_cnt(cnt_ref)
    aggm = (agg_ref[0][:N] + agg_ref[1][:N]) * inv
    h = jnp.maximum(
        lax.dot_general(aggm, w1l_ref[...], (((1,), (1,)), ((), ())),
                        preferred_element_type=jnp.float32) + z1_ref[...],
        0.0)
    h_ref[...] = h
    z2_ref[...] = lax.dot_general(
        h, w2r_ref[...], (((1,), (1,)), ((), ())),
        preferred_element_type=jnp.float32) + b2_ref[...]


_tc_mid = pl.pallas_call(
    _mid_body,
    out_shape=[jax.ShapeDtypeStruct((N, D), jnp.float32),
               jax.ShapeDtypeStruct((N, D), jnp.float32)],
)


def _final_body(agg_ref, cnt_ref, z2_ref, w2l_ref, wlin_ref, blin_ref,
                out_ref):
    inv = _inv_from_cnt(cnt_ref)
    aggm = (agg_ref[0][:N] + agg_ref[1][:N]) * inv
    h2 = jnp.maximum(
        lax.dot_general(aggm, w2l_ref[...], (((1,), (1,)), ((), ())),
                        preferred_element_type=jnp.float32) + z2_ref[...],
        0.0)
    out_ref[...] = lax.dot_general(
        h2, wlin_ref[...], (((1,), (1,)), ((), ())),
        preferred_element_type=jnp.float32) + blin_ref[...]


_tc_final = pl.pallas_call(
    _final_body,
    out_shape=jax.ShapeDtypeStruct((N, 2), jnp.float32),
)


@jax.jit
def kernel(x, edge_index, W1_l, b1_l, W1_r, W2_l, b2_l, W2_r, W_lin, b_lin):
    sc_agg = _make_sc_agg(D)
    sc_counts = _make_sc_counts()

    src = edge_index[0].astype(jnp.int32)
    dst = edge_index[1].astype(jnp.int32)
    pad = E_PAD - E
    srcs = jnp.concatenate([src, jnp.zeros((pad,), jnp.int32)])
    dsts = jnp.concatenate([dst, jnp.full((pad,), N, jnp.int32)])
    srcs = srcs.reshape(NW, CHUNKS_PER_W, CHUNK)
    dsts_c = dsts.reshape(NW, CPW_C, CHUNK_C)
    dsts = dsts.reshape(NW, CHUNKS_PER_W, CHUNK)

    cnt = sc_counts(dsts_c)
    agg1 = sc_agg(_tc_copy(x), srcs, dsts)
    z1 = _tc_affine(x, W1_r, b1_l.reshape(1, D))

    h, z2 = _tc_mid(agg1, cnt, z1, W1_l, W2_r, b2_l.reshape(1, D))

    agg2 = sc_agg(h, srcs, dsts)

    return _tc_final(agg2, cnt, z2, W2_l, W_lin, b_lin.reshape(1, 2))
